# Initial kernel scaffold; baseline (speedup 1.0000x reference)
import jax
import jax.numpy as jnp
from jax.experimental import pallas as pl


def kernel(hidden_states, gate_w, gate_up_proj, down_proj):
    raise NotImplementedError("write your pallas kernel here")


# speedup vs baseline: 2.2292x; 2.2292x over previous
import jax
import jax.numpy as jnp
from jax.experimental import pallas as pl
from jax.experimental.pallas import tpu as pltpu

NE = 8
DM = 768
DF = 768
TM = 1024


def _moe_body(x_ref, gw_ref, gu_ref, dn_ref, out_ref, sc_ref):
    t = pl.program_id(0)
    e = pl.program_id(1)

    @pl.when(e == 0)
    def _router():
        x = x_ref[...]
        logits = jnp.dot(x, gw_ref[...], preferred_element_type=jnp.float32)
        iota = jax.lax.broadcasted_iota(jnp.int32, logits.shape, 1)
        m1 = jnp.max(logits, axis=1, keepdims=True)
        a1 = jnp.min(jnp.where(logits == m1, iota, NE), axis=1, keepdims=True)
        masked = jnp.where(iota == a1, -jnp.inf, logits)
        m2 = jnp.max(masked, axis=1, keepdims=True)
        a2 = jnp.min(jnp.where(masked == m2, iota, NE), axis=1, keepdims=True)
        w0 = 1.0 / (1.0 + jnp.exp(m2 - m1))
        w1 = 1.0 - w0
        sc_ref[...] = (jnp.where(iota == a1, w0, 0.0)
                       + jnp.where(iota == a2, w1, 0.0))
        out_ref[...] = jnp.zeros_like(out_ref)

    x = x_ref[...]
    gu = jnp.dot(x, gu_ref[0], preferred_element_type=jnp.float32)
    g = gu[:, :DF]
    u = gu[:, DF:]
    h = u * (g * jax.nn.sigmoid(g))
    y = jnp.dot(h, dn_ref[0], preferred_element_type=jnp.float32)
    eiota = jax.lax.broadcasted_iota(jnp.int32, (TM, NE), 1)
    w = jnp.sum(jnp.where(eiota == e, sc_ref[...], 0.0), axis=1, keepdims=True)
    out_ref[...] += y * w


def kernel(hidden_states, gate_w, gate_up_proj, down_proj):
    B, S, D = hidden_states.shape
    bs = B * S
    x = hidden_states.reshape(bs, D)
    out = pl.pallas_call(
        _moe_body,
        grid=(bs // TM, NE),
        in_specs=[
            pl.BlockSpec((TM, DM), lambda t, e: (t, 0)),
            pl.BlockSpec((DM, NE), lambda t, e: (0, 0)),
            pl.BlockSpec((1, DM, 2 * DF), lambda t, e: (e, 0, 0)),
            pl.BlockSpec((1, DF, DM), lambda t, e: (e, 0, 0)),
        ],
        out_specs=pl.BlockSpec((TM, DM), lambda t, e: (t, 0)),
        out_shape=jax.ShapeDtypeStruct((bs, DM), jnp.float32),
        scratch_shapes=[pltpu.VMEM((TM, NE), jnp.float32)],
    )(x, gate_w, gate_up_proj, down_proj)
    return out.reshape(B, S, D)
